# baseline (speedup 1.0000x reference)
import functools

import jax
import jax.numpy as jnp
from jax import lax
from jax.experimental import pallas as pl
from jax.experimental.pallas import tpu as pltpu
from jax.experimental.pallas import tpu_sc as plsc

_B = 16384
_D = 64
_F = 32
_H = 64
_NN = 1000000
_BW = 8192
_SH = 13
_NBLK = 31
_V4 = _NBLK * _BW
_LASTBLK = _NN // _BW
_NC = 2
_NS = 16
_NW = _NC * _NS
_BPW = _B // _NW
_CHUNK = 64
_K = _BPW // _CHUNK


def _repack_body(a_ref, b_ref, c_ref, d_ref, out_ref):
    r = lax.broadcasted_iota(jnp.int32, (4 * _D, 4 * _D), 0)
    c = lax.broadcasted_iota(jnp.int32, (4 * _D, 4 * _D), 1)
    ci = c & 127
    perm = ((ci >> 5) << 6) + ((ci & 31) << 1) + (c >> 7)
    em = jnp.float32(1) * (r == perm)
    astack = jnp.concatenate(
        [a_ref[...], b_ref[...], c_ref[...], d_ref[...]], axis=0)
    t = lax.dot_general(astack, em, (((0,), (0,)), ((), ())),
                        preferred_element_type=jnp.float32)
    lo = lax.bitcast_convert_type(
        t[:, : 2 * _D].astype(jnp.bfloat16), jnp.uint16).astype(jnp.uint32)
    hi = lax.bitcast_convert_type(
        t[:, 2 * _D:].astype(jnp.bfloat16), jnp.uint16).astype(jnp.uint32)
    out_ref[...] = lax.bitcast_convert_type(lo | (hi << 16), jnp.float32)


_repack = pl.pallas_call(
    _repack_body,
    grid=(_NBLK,),
    in_specs=[
        pl.BlockSpec((_D, _BW),
                     lambda i, k=k: (0, jnp.minimum(4 * i + k, _LASTBLK)))
        for k in range(4)
    ],
    out_specs=pl.BlockSpec((_BW, 2 * _D), lambda i: (i, 0)),
    out_shape=jax.ShapeDtypeStruct((_V4, 2 * _D), jnp.float32),
    compiler_params=pltpu.CompilerParams(
        dimension_semantics=("parallel",),
    ),
)


def _sc_gather_body(table_hbm, idx_hbm, out_hbm, idx_v, rows_v, sem, wsem):
    wid = lax.axis_index("s") * _NC + lax.axis_index("c")
    pltpu.sync_copy(idx_hbm.at[wid], idx_v)
    copies = [
        pltpu.async_copy(
            table_hbm.at[idx_v.at[j]],
            rows_v.at[pl.ds(j * _CHUNK, _CHUNK)],
            sem,
        )
        for j in range(_K)
    ]
    wcopies = []
    for j, c in enumerate(copies):
        c.wait()
        wcopies.append(pltpu.async_copy(
            rows_v.at[pl.ds(j * _CHUNK, _CHUNK)],
            out_hbm.at[pl.ds(wid * _BPW + j * _CHUNK, _CHUNK)],
            wsem,
        ))
    for w in wcopies:
        w.wait()


_sc_gather = functools.partial(
    pl.kernel,
    mesh=plsc.VectorSubcoreMesh(core_axis_name="c", subcore_axis_name="s"),
    out_type=jax.ShapeDtypeStruct((_B, 2 * _D), jnp.float32),
    scratch_types=[
        pltpu.VMEM((_K, _CHUNK), jnp.int32),
        pltpu.VMEM((_BPW, 2 * _D), jnp.float32),
        pltpu.SemaphoreType.DMA,
        pltpu.SemaphoreType.DMA,
    ],
    compiler_params=pltpu.CompilerParams(use_tc_tiling_on_sc=True),
)(_sc_gather_body)


_R = 2048


def _mlp_body(g_ref, featt_ref, quad_ref, w1p_ref, b1_ref,
              w2_ref, b2_ref, out_ref):
    w1me = w1p_ref[: _F, :]
    w1mo = w1p_ref[_F: 2 * _F, :]
    w1f = w1p_ref[2 * _F:, :]
    w1me4 = jnp.concatenate([w1me, w1me, w1me, w1me], axis=0)
    w1mo4 = jnp.concatenate([w1mo, w1mo, w1mo, w1mo], axis=0)
    w = lax.bitcast_convert_type(g_ref[...], jnp.uint32)
    ge = lax.bitcast_convert_type((w & 0xFFFF).astype(jnp.uint16),
                                  jnp.bfloat16)
    go = lax.bitcast_convert_type((w >> 16).astype(jnp.uint16), jnp.bfloat16)
    lane_q = (lax.broadcasted_iota(jnp.int32, (1, 2 * _D), 1) >> 5).astype(
        jnp.float32)
    quad = quad_ref[...].reshape(_R, 1)
    mask = (jnp.float32(1) * (lane_q == quad)).astype(jnp.bfloat16)
    hm = (jnp.dot(ge * mask, w1me4.astype(jnp.bfloat16),
                  preferred_element_type=jnp.float32)
          + jnp.dot(go * mask, w1mo4.astype(jnp.bfloat16),
                    preferred_element_type=jnp.float32))
    hf = lax.dot_general(featt_ref[...], w1f, (((0,), (0,)), ((), ())),
                         preferred_element_type=jnp.float32)
    h = jnp.maximum(hm + hf + b1_ref[...].reshape(1, _H), 0.0)
    out = (jnp.dot(h, w2_ref[...], preferred_element_type=jnp.float32)
           + b2_ref[...].reshape(1, 1))
    out_ref[...] = out.reshape(_R)


_mlp = pl.pallas_call(
    _mlp_body,
    grid=(_B // _R,),
    in_specs=[
        pl.BlockSpec((_R, 2 * _D), lambda i: (i, 0)),
        pl.BlockSpec((_F, _R), lambda i: (0, i)),
        pl.BlockSpec((_R,), lambda i: (i,)),
        pl.BlockSpec((3 * _F, _H), lambda i: (0, 0)),
        pl.BlockSpec((_H,), lambda i: (0,)),
        pl.BlockSpec((_H, 1), lambda i: (0, 0)),
        pl.BlockSpec((1,), lambda i: (0,)),
    ],
    out_specs=pl.BlockSpec((_R,), lambda i: (i,)),
    out_shape=jax.ShapeDtypeStruct((_B,), jnp.float32),
    compiler_params=pltpu.CompilerParams(
        dimension_semantics=("parallel",),
    ),
)


def kernel(n_id, node_features_at_t, memory, W1, b1, W2, b2):
    packed = _repack(memory.T, memory.T, memory.T, memory.T)
    n = n_id.astype(jnp.int32)
    ib = n >> _SH
    pos = n & (_BW - 1)
    row = ((ib >> 2) << _SH) | pos
    quad = (ib & 3).astype(jnp.float32)
    idx = row.reshape(_NW, _K, _CHUNK)
    g = _sc_gather(packed, idx)
    w1p = jnp.concatenate([W1[0:_D:2], W1[1:_D:2], W1[_D:]], axis=0)
    out = _mlp(g, node_features_at_t.T, quad, w1p, b1, W2, b2)
    return out.reshape(_B, 1)


# speedup vs baseline: 1.0085x; 1.0085x over previous
import functools

import jax
import jax.numpy as jnp
from jax import lax
from jax.experimental import pallas as pl
from jax.experimental.pallas import tpu as pltpu
from jax.experimental.pallas import tpu_sc as plsc

_B = 16384
_D = 64
_F = 32
_H = 64
_NN = 1000000
_BW = 8192
_SH = 13
_NBLK = 31
_V4 = _NBLK * _BW
_LASTBLK = _NN // _BW
_NC = 2
_NS = 16
_NW = _NC * _NS
_BPW = _B // _NW
_CHUNK = 64
_K = _BPW // _CHUNK


def _repack_body(a_ref, b_ref, c_ref, d_ref, out_ref):
    r = lax.broadcasted_iota(jnp.int32, (4 * _D, 4 * _D), 0)
    c = lax.broadcasted_iota(jnp.int32, (4 * _D, 4 * _D), 1)
    ci = c & 127
    perm = ((ci >> 5) << 6) + ((ci & 31) << 1) + (c >> 7)
    em = jnp.float32(1) * (r == perm)
    astack = jnp.concatenate(
        [a_ref[...], b_ref[...], c_ref[...], d_ref[...]], axis=0)
    t = lax.dot_general(astack, em, (((0,), (0,)), ((), ())),
                        preferred_element_type=jnp.float32)
    lo = lax.bitcast_convert_type(
        t[:, : 2 * _D].astype(jnp.bfloat16), jnp.uint16).astype(jnp.uint32)
    hi = lax.bitcast_convert_type(
        t[:, 2 * _D:].astype(jnp.bfloat16), jnp.uint16).astype(jnp.uint32)
    out_ref[...] = lax.bitcast_convert_type(lo | (hi << 16), jnp.float32)


_repack = pl.pallas_call(
    _repack_body,
    grid=(_NBLK,),
    in_specs=[
        pl.BlockSpec((_D, _BW),
                     lambda i, k=k: (0, jnp.minimum(4 * i + k, _LASTBLK)))
        for k in range(4)
    ],
    out_specs=pl.BlockSpec((_BW, 2 * _D), lambda i: (i, 0)),
    out_shape=jax.ShapeDtypeStruct((_V4, 2 * _D), jnp.float32),
    compiler_params=pltpu.CompilerParams(
        dimension_semantics=("parallel",),
    ),
)


def _sc_gather_body(table_hbm, idx_hbm, out_hbm, idx_v, rows_v, sem, wsem):
    wid = lax.axis_index("s") * _NC + lax.axis_index("c")
    pltpu.sync_copy(idx_hbm.at[wid], idx_v)
    copies = [
        pltpu.async_copy(
            table_hbm.at[idx_v.at[j]],
            rows_v.at[pl.ds(j * _CHUNK, _CHUNK)],
            sem,
        )
        for j in range(_K)
    ]
    wcopies = []
    for j, c in enumerate(copies):
        c.wait()
        wcopies.append(pltpu.async_copy(
            rows_v.at[pl.ds(j * _CHUNK, _CHUNK)],
            out_hbm.at[pl.ds(wid * _BPW + j * _CHUNK, _CHUNK)],
            wsem,
        ))
    for w in wcopies:
        w.wait()


_sc_gather = functools.partial(
    pl.kernel,
    mesh=plsc.VectorSubcoreMesh(core_axis_name="c", subcore_axis_name="s"),
    out_type=jax.ShapeDtypeStruct((_B, 2 * _D), jnp.float32),
    scratch_types=[
        pltpu.VMEM((_K, _CHUNK), jnp.int32),
        pltpu.VMEM((_BPW, 2 * _D), jnp.float32),
        pltpu.SemaphoreType.DMA,
        pltpu.SemaphoreType.DMA,
    ],
    compiler_params=pltpu.CompilerParams(use_tc_tiling_on_sc=True),
)(_sc_gather_body)


_R = 4096


def _mlp_body(g_ref, featt_ref, quad_ref, w1p_ref, b1_ref,
              w2_ref, b2_ref, out_ref):
    w1me = w1p_ref[: _F, :]
    w1mo = w1p_ref[_F: 2 * _F, :]
    w1f = w1p_ref[2 * _F:, :]
    w1me4 = jnp.concatenate([w1me, w1me, w1me, w1me], axis=0)
    w1mo4 = jnp.concatenate([w1mo, w1mo, w1mo, w1mo], axis=0)
    w = lax.bitcast_convert_type(g_ref[...], jnp.uint32)
    ge = lax.bitcast_convert_type((w & 0xFFFF).astype(jnp.uint16),
                                  jnp.bfloat16)
    go = lax.bitcast_convert_type((w >> 16).astype(jnp.uint16), jnp.bfloat16)
    lane_q = (lax.broadcasted_iota(jnp.int32, (1, 2 * _D), 1) >> 5).astype(
        jnp.float32)
    quad = quad_ref[...].reshape(_R, 1)
    mask = (jnp.float32(1) * (lane_q == quad)).astype(jnp.bfloat16)
    hm = (jnp.dot(ge * mask, w1me4.astype(jnp.bfloat16),
                  preferred_element_type=jnp.float32)
          + jnp.dot(go * mask, w1mo4.astype(jnp.bfloat16),
                    preferred_element_type=jnp.float32))
    hf = lax.dot_general(featt_ref[...], w1f, (((0,), (0,)), ((), ())),
                         preferred_element_type=jnp.float32)
    h = jnp.maximum(hm + hf + b1_ref[...].reshape(1, _H), 0.0)
    out = (jnp.dot(h, w2_ref[...], preferred_element_type=jnp.float32)
           + b2_ref[...].reshape(1, 1))
    out_ref[...] = out.reshape(_R)


_mlp = pl.pallas_call(
    _mlp_body,
    grid=(_B // _R,),
    in_specs=[
        pl.BlockSpec((_R, 2 * _D), lambda i: (i, 0)),
        pl.BlockSpec((_F, _R), lambda i: (0, i)),
        pl.BlockSpec((_R,), lambda i: (i,)),
        pl.BlockSpec((3 * _F, _H), lambda i: (0, 0)),
        pl.BlockSpec((_H,), lambda i: (0,)),
        pl.BlockSpec((_H, 1), lambda i: (0, 0)),
        pl.BlockSpec((1,), lambda i: (0,)),
    ],
    out_specs=pl.BlockSpec((_R,), lambda i: (i,)),
    out_shape=jax.ShapeDtypeStruct((_B,), jnp.float32),
    compiler_params=pltpu.CompilerParams(
        dimension_semantics=("parallel",),
    ),
)


def kernel(n_id, node_features_at_t, memory, W1, b1, W2, b2):
    packed = _repack(memory.T, memory.T, memory.T, memory.T)
    n = n_id.astype(jnp.int32)
    ib = n >> _SH
    pos = n & (_BW - 1)
    row = ((ib >> 2) << _SH) | pos
    quad = (ib & 3).astype(jnp.float32)
    idx = row.reshape(_NW, _K, _CHUNK)
    g = _sc_gather(packed, idx)
    w1p = jnp.concatenate([W1[0:_D:2], W1[1:_D:2], W1[_D:]], axis=0)
    out = _mlp(g, node_features_at_t.T, quad, w1p, b1, W2, b2)
    return out.reshape(_B, 1)
